# baseline (speedup 1.0000x reference)
import jax
import jax.numpy as jnp
from jax.experimental import pallas as pl


def _fps(xyz, npoint):
    Bn, Nn = xyz.shape[0], xyz.shape[1]
    dists0 = jnp.full((Bn, Nn), 1e10, jnp.float32)
    far0 = jnp.zeros((Bn,), jnp.int32)

    def step(carry, _):
        dists, far = carry
        centroid = xyz[jnp.arange(Bn), far][:, None, :]
        d = jnp.sum((xyz - centroid) ** 2, -1)
        dists = jnp.minimum(dists, d)
        nxt = jnp.argmax(dists, axis=1).astype(jnp.int32)
        return (dists, nxt), far

    _, idxs = jax.lax.scan(step, (dists0, far0), None, length=npoint)
    return jnp.transpose(idxs)


def _index_points(points, idx):
    Bn = points.shape[0]
    bidx = jnp.arange(Bn).reshape((Bn,) + (1,) * (idx.ndim - 1))
    return points[bidx, idx]


def _ball_query(radius, nsample, xyz, new_xyz):
    Bn, S = new_xyz.shape[0], new_xyz.shape[1]
    Nn = xyz.shape[1]
    sq = jnp.sum((new_xyz[:, :, None, :] - xyz[:, None, :, :]) ** 2, -1)
    gi = jnp.broadcast_to(jnp.arange(Nn, dtype=jnp.int32), (Bn, S, Nn))
    gi = jnp.where(sq > radius * radius, Nn, gi)
    gi = jnp.sort(gi, axis=-1)[:, :, :nsample]
    first = gi[:, :, :1]
    gi = jnp.where(gi == Nn, first, gi)
    gi = jnp.minimum(gi, Nn - 1)
    return gi


def _set_conv(xyz, feat, npoint, radius, nsample, Ws, bs):
    fi = _fps(xyz, npoint)
    nx = _index_points(xyz, fi)
    gi = _ball_query(radius, nsample, xyz, nx)
    gx = _index_points(xyz, gi) - nx[:, :, None, :]
    gf = _index_points(feat, gi)
    g = jnp.concatenate([gx, gf], -1)
    for W, b in zip(Ws, bs):
        g = jax.nn.relu(g @ W + b)
    return jnp.max(g, axis=2), nx, fi


def _set_upconv(xyz_s, xyz_d, feat_s, feat_d, radius, nsample, W1s, b1s, W2s, b2s):
    gi = _ball_query(radius, nsample, xyz_s, xyz_d)
    gx = _index_points(xyz_s, gi) - xyz_d[:, :, None, :]
    gf = _index_points(feat_s, gi)
    g = jnp.concatenate([gx, gf], -1)
    for W, b in zip(W1s, b1s):
        g = jax.nn.relu(g @ W + b)
    g = jnp.max(g, axis=2)
    g = jnp.concatenate([g, feat_d], -1)
    for W, b in zip(W2s, b2s):
        g = jax.nn.relu(g @ W + b)
    return g


def kernel(pc, feat, down0_W, down0_b, d1_W0, d1_b0, d1_W1, d1_b1, d2_W0, d2_b0, d2_W1, d2_b1, d3_W0, d3_b0, d3_W1, d3_b1, d4_W0, d4_b0, d4_W1, d4_b1, u4_W1, u4_b1, u4_W2, u4_b2, u3_W1, u3_b1, u3_W2, u3_b2, u2_W1, u2_b1, u2_W2, u2_b2, u1_W1, u1_b1, u1_W2, u1_b2):
    f0 = jax.nn.relu(feat @ down0_W + down0_b)
    f1, x1, i1 = _set_conv(pc, f0, 2048, 0.5, 16, [d1_W0, d1_W1], [d1_b0, d1_b1])
    f2, x2, i2 = _set_conv(x1, f1, 512, 1.0, 16, [d2_W0, d2_W1], [d2_b0, d2_b1])
    f3, x3, i3 = _set_conv(x2, f2, 128, 2.0, 16, [d3_W0, d3_W1], [d3_b0, d3_b1])
    f4, x4, i4 = _set_conv(x3, f3, 64, 4.0, 16, [d4_W0, d4_W1], [d4_b0, d4_b1])
    u3 = _set_upconv(x4, x3, f4, f3, 6.0, 8, [u4_W1], [u4_b1], [u4_W2], [u4_b2])
    u2 = _set_upconv(x3, x2, u3, f2, 3.0, 8, [u3_W1], [u3_b1], [u3_W2], [u3_b2])
    u1 = _set_upconv(x2, x1, u2, f1, 1.5, 8, [u2_W1], [u2_b1], [u2_W2], [u2_b2])
    u0 = _set_upconv(x1, pc, u1, f0, 0.75, 8, [u1_W1], [u1_b1], [u1_W2], [u1_b2])
    return (x1, x2, x3, x4), (i1, i2, i3, i4), (u0, u1, u2, u3)


# speedup vs baseline: 9.6141x; 9.6141x over previous
import functools

import jax
import jax.numpy as jnp
from jax.experimental import pallas as pl
from jax.experimental.pallas import tpu as pltpu


def _call(*args, **kwargs):
    return pl.pallas_call(*args, **kwargs)


def _rup(x, m):
    return ((x + m - 1) // m) * m


def _pad_last(a, to):
    if a.shape[-1] == to:
        return a
    pad = [(0, 0)] * (a.ndim - 1) + [(0, to - a.shape[-1])]
    return jnp.pad(a, pad)


def _fps_kern(npoint, N, NL, x_ref, idx_ref, nx_ref, dists):
    X = x_ref[0, 0]
    Y = x_ref[0, 1]
    Z = x_ref[0, 2]
    iota2 = (jax.lax.broadcasted_iota(jnp.int32, (8, NL), 0) * NL
             + jax.lax.broadcasted_iota(jnp.int32, (8, NL), 1))
    lane = jax.lax.broadcasted_iota(jnp.int32, (1, 128), 1)
    dists[...] = jnp.full((8, NL), 1e10, jnp.float32)

    def step(t, far):
        ohf = (iota2 == far).astype(jnp.float32)
        cx = jnp.sum(X * ohf)
        cy = jnp.sum(Y * ohf)
        cz = jnp.sum(Z * ohf)
        idx_ref[0, pl.ds(t, 1), :] = jnp.full((1, 128), far, jnp.int32)
        nxrow = jnp.where(lane == 0, cx, jnp.where(lane == 1, cy,
                          jnp.where(lane == 2, cz, 0.0))).astype(jnp.float32)
        nx_ref[0, pl.ds(t, 1), :] = nxrow
        dx = X - cx
        dy = Y - cy
        dz = Z - cz
        d = (dx * dx + dy * dy) + dz * dz
        nd = jnp.minimum(dists[...], d)
        dists[...] = nd
        m = jnp.max(nd)
        return jnp.min(jnp.where(nd == m, iota2, N)).astype(jnp.int32)

    jax.lax.fori_loop(0, npoint, step, jnp.int32(0))


def _fps(xyz, npoint):
    B, N, _ = xyz.shape
    NL = N // 8
    xr = xyz.transpose(0, 2, 1).reshape(B, 3, 8, NL)
    idx, nx = _call(
        functools.partial(_fps_kern, npoint, N, NL),
        grid=(B,),
        in_specs=[pl.BlockSpec((1, 3, 8, NL), lambda b: (b, 0, 0, 0))],
        out_specs=[pl.BlockSpec((1, npoint, 128), lambda b: (b, 0, 0)),
                   pl.BlockSpec((1, npoint, 128), lambda b: (b, 0, 0))],
        out_shape=[jax.ShapeDtypeStruct((B, npoint, 128), jnp.int32),
                   jax.ShapeDtypeStruct((B, npoint, 128), jnp.float32)],
        scratch_shapes=[pltpu.VMEM((8, NL), jnp.float32)],
    )(xr)
    return idx[:, :, 0], nx[:, :, :3]


def _conv_kern(N, SB, r2, ns, nW1, has_fd, refs):
    if has_fd:
        (xyzT_ref, q_ref, F_ref, W0_ref, b0_ref, W1_ref, b1_ref,
         fd_ref, W2a_ref, W2b_ref, b2_ref, o_ref) = refs
    else:
        (xyzT_ref, q_ref, F_ref, W0_ref, b0_ref, W1_ref, b1_ref,
         o_ref) = refs
    q = q_ref[0]
    xx = xyzT_ref[0, 0:1, :]
    xy = xyzT_ref[0, 1:2, :]
    xz = xyzT_ref[0, 2:3, :]
    dx = q[:, 0:1] - xx
    dy = q[:, 1:2] - xy
    dz = q[:, 2:3] - xz
    sq = (dx * dx + dy * dy) + dz * dz
    ioN = jax.lax.broadcasted_iota(jnp.int32, (SB, N), 1)
    key = jnp.where(sq > r2, N, ioN)
    cols = []
    for _ in range(ns):
        cur = jnp.min(key, axis=1, keepdims=True)
        cols.append(cur)
        key = jnp.where(key == cur, N, key)
    first = cols[0]
    Fb = F_ref[0]
    W0 = W0_ref[...]
    b0 = b0_ref[...]
    W1 = W1_ref[...]
    b1 = b1_ref[...]
    acc = None
    for j in range(ns):
        nbr = jnp.minimum(jnp.where(cols[j] == N, first, cols[j]), N - 1)
        ohj = (ioN == nbr).astype(jnp.float32)
        g = jax.lax.dot_general(ohj, Fb, (((1,), (0,)), ((), ())),
                                preferred_element_type=jnp.float32)
        g = g - q
        h = jnp.maximum(
            jax.lax.dot_general(g, W0, (((1,), (0,)), ((), ())),
                                preferred_element_type=jnp.float32) + b0, 0.0)
        if nW1:
            h = jnp.maximum(
                jax.lax.dot_general(h, W1, (((1,), (0,)), ((), ())),
                                    preferred_element_type=jnp.float32) + b1,
                0.0)
        acc = h if acc is None else jnp.maximum(acc, h)
    if has_fd:
        fd = fd_ref[0]
        o = jnp.maximum(
            jax.lax.dot_general(acc, W2a_ref[...], (((1,), (0,)), ((), ())),
                                preferred_element_type=jnp.float32)
            + jax.lax.dot_general(fd, W2b_ref[...], (((1,), (0,)), ((), ())),
                                  preferred_element_type=jnp.float32)
            + b2_ref[...], 0.0)
    else:
        o = acc
    o_ref[0] = o


def _conv_level(xyz, new_xyz, feat, radius, nsample, Ws, bs, SB,
                feat_d=None, W2=None, b2=None):
    B, N, _ = xyz.shape
    S = new_xyz.shape[1]
    Cf = feat.shape[-1]
    Cin = 3 + Cf
    CP = _rup(Cin, 128)
    H1 = Ws[0].shape[1]
    H1P = _rup(H1, 128)
    nW1 = len(Ws) > 1
    if nW1:
        H2 = Ws[1].shape[1]
        H2P = _rup(H2, 128)
    else:
        H2, H2P = H1, H1P
    has_fd = feat_d is not None

    xyzT = _pad_last(xyz, 8).transpose(0, 2, 1)
    q_pad = _pad_last(new_xyz, CP)
    F_all = _pad_last(jnp.concatenate([xyz, feat], -1), CP)
    W0p = _pad_last(jnp.pad(Ws[0], ((0, CP - Cin), (0, 0))), H1P)
    b0p = _pad_last(bs[0][None, :], H1P)
    if nW1:
        W1p = _pad_last(jnp.pad(Ws[1], ((0, H1P - H1), (0, 0))), H2P)
        b1p = _pad_last(bs[1][None, :], H2P)
    else:
        W1p = jnp.zeros((8, 128), jnp.float32)
        b1p = jnp.zeros((1, 128), jnp.float32)

    r2 = float(radius) * float(radius)
    nsb = S // SB
    grid = (B, nsb)
    in_specs = [
        pl.BlockSpec((1, 8, N), lambda b, s: (b, 0, 0)),
        pl.BlockSpec((1, SB, CP), lambda b, s: (b, s, 0)),
        pl.BlockSpec((1, N, CP), lambda b, s: (b, 0, 0)),
        pl.BlockSpec(W0p.shape, lambda b, s: (0, 0)),
        pl.BlockSpec(b0p.shape, lambda b, s: (0, 0)),
        pl.BlockSpec(W1p.shape, lambda b, s: (0, 0)),
        pl.BlockSpec(b1p.shape, lambda b, s: (0, 0)),
    ]
    args = [xyzT, q_pad, F_all, W0p, b0p, W1p, b1p]
    if has_fd:
        Cd = feat_d.shape[-1]
        CdP = _rup(Cd, 128)
        H3 = W2.shape[1]
        H3P = _rup(H3, 128)
        fdp = _pad_last(feat_d, CdP)
        W2a = _pad_last(jnp.pad(W2[:H2], ((0, H2P - H2), (0, 0))), H3P)
        W2b = _pad_last(jnp.pad(W2[H2:], ((0, CdP - Cd), (0, 0))), H3P)
        b2p = _pad_last(b2[None, :], H3P)
        in_specs += [
            pl.BlockSpec((1, SB, CdP), lambda b, s: (b, s, 0)),
            pl.BlockSpec(W2a.shape, lambda b, s: (0, 0)),
            pl.BlockSpec(W2b.shape, lambda b, s: (0, 0)),
            pl.BlockSpec(b2p.shape, lambda b, s: (0, 0)),
        ]
        args += [fdp, W2a, W2b, b2p]
        HoutP = H3P
    else:
        HoutP = H2P

    def kern(*refs):
        _conv_kern(N, SB, r2, nsample, nW1, has_fd, refs)

    out = _call(
        kern,
        grid=grid,
        in_specs=in_specs,
        out_specs=pl.BlockSpec((1, SB, HoutP), lambda b, s: (b, s, 0)),
        out_shape=jax.ShapeDtypeStruct((B, S, HoutP), jnp.float32),
    )(*args)
    return out


def _down0_kern(x_ref, W_ref, b_ref, o_ref):
    o_ref[0] = jnp.maximum(
        jax.lax.dot_general(x_ref[0], W_ref[...], (((1,), (0,)), ((), ())),
                            preferred_element_type=jnp.float32) + b_ref[...],
        0.0)


def _down0(feat, W, b):
    B, N, C = feat.shape
    H = W.shape[1]
    HP = _rup(H, 128)
    xp = _pad_last(feat, 8)
    Wp = _pad_last(jnp.pad(W, ((0, 8 - C), (0, 0))), HP)
    bp = _pad_last(b[None, :], HP)
    out = _call(
        _down0_kern,
        grid=(B,),
        in_specs=[pl.BlockSpec((1, N, 8), lambda bb: (bb, 0, 0)),
                  pl.BlockSpec(Wp.shape, lambda bb: (0, 0)),
                  pl.BlockSpec(bp.shape, lambda bb: (0, 0))],
        out_specs=pl.BlockSpec((1, N, HP), lambda bb: (bb, 0, 0)),
        out_shape=jax.ShapeDtypeStruct((B, N, HP), jnp.float32),
    )(xp, Wp, bp)
    return out


def kernel(pc, feat, down0_W, down0_b, d1_W0, d1_b0, d1_W1, d1_b1,
           d2_W0, d2_b0, d2_W1, d2_b1, d3_W0, d3_b0, d3_W1, d3_b1,
           d4_W0, d4_b0, d4_W1, d4_b1, u4_W1, u4_b1, u4_W2, u4_b2,
           u3_W1, u3_b1, u3_W2, u3_b2, u2_W1, u2_b1, u2_W2, u2_b2,
           u1_W1, u1_b1, u1_W2, u1_b2):
    f0p = _down0(feat, down0_W, down0_b)
    f0 = f0p[:, :, :32]

    i1, x1 = _fps(pc, 2048)
    f1 = _conv_level(pc, x1, f0, 0.5, 16, [d1_W0, d1_W1], [d1_b0, d1_b1],
                     SB=128)[:, :, :64]
    i2, x2 = _fps(x1, 512)
    f2 = _conv_level(x1, x2, f1, 1.0, 16, [d2_W0, d2_W1], [d2_b0, d2_b1],
                     SB=512)[:, :, :128]
    i3, x3 = _fps(x2, 128)
    f3 = _conv_level(x2, x3, f2, 2.0, 16, [d3_W0, d3_W1], [d3_b0, d3_b1],
                     SB=128)[:, :, :192]
    i4, x4 = _fps(x3, 64)
    f4 = _conv_level(x3, x4, f3, 4.0, 16, [d4_W0, d4_W1], [d4_b0, d4_b1],
                     SB=64)[:, :, :192]

    u3 = _conv_level(x4, x3, f4, 6.0, 8, [u4_W1], [u4_b1], SB=128,
                     feat_d=f3, W2=u4_W2, b2=u4_b2)[:, :, :192]
    u2 = _conv_level(x3, x2, u3, 3.0, 8, [u3_W1], [u3_b1], SB=512,
                     feat_d=f2, W2=u3_W2, b2=u3_b2)[:, :, :128]
    u1 = _conv_level(x2, x1, u2, 1.5, 8, [u2_W1], [u2_b1], SB=512,
                     feat_d=f1, W2=u2_W2, b2=u2_b2)[:, :, :64]
    u0 = _conv_level(x1, pc, u1, 0.75, 8, [u1_W1], [u1_b1], SB=512,
                     feat_d=f0, W2=u1_W2, b2=u1_b2)[:, :, :32]

    return (x1, x2, x3, x4), (i1, i2, i3, i4), (u0, u1, u2, u3)


# speedup vs baseline: 9.8012x; 1.0195x over previous
import functools

import jax
import jax.numpy as jnp
from jax.experimental import pallas as pl
from jax.experimental.pallas import tpu as pltpu


def _call(*args, **kwargs):
    return pl.pallas_call(*args, **kwargs)


def _rup(x, m):
    return ((x + m - 1) // m) * m


def _pad_last(a, to):
    if a.shape[-1] == to:
        return a
    pad = [(0, 0)] * (a.ndim - 1) + [(0, to - a.shape[-1])]
    return jnp.pad(a, pad)


def _fps_kern(npoint, N, NL, x_ref, xr_ref, idx_ref, nx_ref, dists):
    X = x_ref[0, 0]
    Y = x_ref[0, 1]
    Z = x_ref[0, 2]
    iota2 = (jax.lax.broadcasted_iota(jnp.int32, (8, NL), 0) * NL
             + jax.lax.broadcasted_iota(jnp.int32, (8, NL), 1))
    dists[...] = jnp.full((8, NL), 1e10, jnp.float32)

    def step(t, far):
        crow = xr_ref[0, pl.ds(far, 1), :]
        cx = crow[0:1, 0:1]
        cy = crow[0:1, 1:2]
        cz = crow[0:1, 2:3]
        idx_ref[0, pl.ds(t, 1), :] = jnp.full((1, 128), far, jnp.int32)
        nx_ref[0, pl.ds(t, 1), :] = crow
        dx = X - cx
        dy = Y - cy
        dz = Z - cz
        d = (dx * dx + dy * dy) + dz * dz
        nd = jnp.minimum(dists[...], d)
        dists[...] = nd
        m = jnp.max(nd)
        return jnp.min(jnp.where(nd == m, iota2, N)).astype(jnp.int32)

    jax.lax.fori_loop(0, npoint, step, jnp.int32(0))


def _fps(xyz, npoint):
    B, N, _ = xyz.shape
    NL = N // 8
    xr = xyz.transpose(0, 2, 1).reshape(B, 3, 8, NL)
    xrows = _pad_last(xyz, 128)
    idx, nx = _call(
        functools.partial(_fps_kern, npoint, N, NL),
        grid=(B,),
        in_specs=[pl.BlockSpec((1, 3, 8, NL), lambda b: (b, 0, 0, 0)),
                  pl.BlockSpec((1, N, 128), lambda b: (b, 0, 0))],
        out_specs=[pl.BlockSpec((1, npoint, 128), lambda b: (b, 0, 0)),
                   pl.BlockSpec((1, npoint, 128), lambda b: (b, 0, 0))],
        out_shape=[jax.ShapeDtypeStruct((B, npoint, 128), jnp.int32),
                   jax.ShapeDtypeStruct((B, npoint, 128), jnp.float32)],
        scratch_shapes=[pltpu.VMEM((8, NL), jnp.float32)],
    )(xr, xrows)
    return idx[:, :, 0], nx[:, :, :3]


def _conv_kern(N, SB, r2, ns, nW1, has_fd, refs):
    if has_fd:
        (xyzT_ref, q_ref, F_ref, W0_ref, b0_ref, W1_ref, b1_ref,
         fd_ref, W2a_ref, W2b_ref, b2_ref, o_ref) = refs
    else:
        (xyzT_ref, q_ref, F_ref, W0_ref, b0_ref, W1_ref, b1_ref,
         o_ref) = refs
    q = q_ref[0]
    xx = xyzT_ref[0, 0:1, :]
    xy = xyzT_ref[0, 1:2, :]
    xz = xyzT_ref[0, 2:3, :]
    dx = q[:, 0:1] - xx
    dy = q[:, 1:2] - xy
    dz = q[:, 2:3] - xz
    sq = (dx * dx + dy * dy) + dz * dz
    ioN = jax.lax.broadcasted_iota(jnp.int32, (SB, N), 1)
    mask = sq <= r2
    rank = mask.astype(jnp.int32)
    k = 1
    while k < N:
        rolled = pltpu.roll(rank, k, axis=1)
        rank = rank + jnp.where(ioN >= k, rolled, 0)
        k *= 2
    cnt = rank[:, N - 1:N]
    ohz = (cnt == 0) & (ioN == N - 1)
    Fb = F_ref[0]
    W0 = W0_ref[...]
    b0 = b0_ref[...]
    W1 = W1_ref[...]
    b1 = b1_ref[...]
    acc = None
    for j in range(ns):
        selj = jnp.where(cnt >= j + 1, j + 1, 1)
        ohj = ((mask & (rank == selj)) | ohz).astype(jnp.float32)
        g = jax.lax.dot_general(ohj, Fb, (((1,), (0,)), ((), ())),
                                preferred_element_type=jnp.float32)
        g = g - q
        h = jnp.maximum(
            jax.lax.dot_general(g, W0, (((1,), (0,)), ((), ())),
                                preferred_element_type=jnp.float32) + b0, 0.0)
        if nW1:
            h = jnp.maximum(
                jax.lax.dot_general(h, W1, (((1,), (0,)), ((), ())),
                                    preferred_element_type=jnp.float32) + b1,
                0.0)
        acc = h if acc is None else jnp.maximum(acc, h)
    if has_fd:
        fd = fd_ref[0]
        o = jnp.maximum(
            jax.lax.dot_general(acc, W2a_ref[...], (((1,), (0,)), ((), ())),
                                preferred_element_type=jnp.float32)
            + jax.lax.dot_general(fd, W2b_ref[...], (((1,), (0,)), ((), ())),
                                  preferred_element_type=jnp.float32)
            + b2_ref[...], 0.0)
    else:
        o = acc
    o_ref[0] = o


def _conv_level(xyz, new_xyz, feat, radius, nsample, Ws, bs, SB,
                feat_d=None, W2=None, b2=None):
    B, N, _ = xyz.shape
    S = new_xyz.shape[1]
    Cf = feat.shape[-1]
    Cin = 3 + Cf
    CP = _rup(Cin, 128)
    H1 = Ws[0].shape[1]
    H1P = _rup(H1, 128)
    nW1 = len(Ws) > 1
    if nW1:
        H2 = Ws[1].shape[1]
        H2P = _rup(H2, 128)
    else:
        H2, H2P = H1, H1P
    has_fd = feat_d is not None

    xyzT = _pad_last(xyz, 8).transpose(0, 2, 1)
    q_pad = _pad_last(new_xyz, CP)
    F_all = _pad_last(jnp.concatenate([xyz, feat], -1), CP)
    W0p = _pad_last(jnp.pad(Ws[0], ((0, CP - Cin), (0, 0))), H1P)
    b0p = _pad_last(bs[0][None, :], H1P)
    if nW1:
        W1p = _pad_last(jnp.pad(Ws[1], ((0, H1P - H1), (0, 0))), H2P)
        b1p = _pad_last(bs[1][None, :], H2P)
    else:
        W1p = jnp.zeros((8, 128), jnp.float32)
        b1p = jnp.zeros((1, 128), jnp.float32)

    r2 = float(radius) * float(radius)
    nsb = S // SB
    grid = (B, nsb)
    in_specs = [
        pl.BlockSpec((1, 8, N), lambda b, s: (b, 0, 0)),
        pl.BlockSpec((1, SB, CP), lambda b, s: (b, s, 0)),
        pl.BlockSpec((1, N, CP), lambda b, s: (b, 0, 0)),
        pl.BlockSpec(W0p.shape, lambda b, s: (0, 0)),
        pl.BlockSpec(b0p.shape, lambda b, s: (0, 0)),
        pl.BlockSpec(W1p.shape, lambda b, s: (0, 0)),
        pl.BlockSpec(b1p.shape, lambda b, s: (0, 0)),
    ]
    args = [xyzT, q_pad, F_all, W0p, b0p, W1p, b1p]
    if has_fd:
        Cd = feat_d.shape[-1]
        CdP = _rup(Cd, 128)
        H3 = W2.shape[1]
        H3P = _rup(H3, 128)
        fdp = _pad_last(feat_d, CdP)
        W2a = _pad_last(jnp.pad(W2[:H2], ((0, H2P - H2), (0, 0))), H3P)
        W2b = _pad_last(jnp.pad(W2[H2:], ((0, CdP - Cd), (0, 0))), H3P)
        b2p = _pad_last(b2[None, :], H3P)
        in_specs += [
            pl.BlockSpec((1, SB, CdP), lambda b, s: (b, s, 0)),
            pl.BlockSpec(W2a.shape, lambda b, s: (0, 0)),
            pl.BlockSpec(W2b.shape, lambda b, s: (0, 0)),
            pl.BlockSpec(b2p.shape, lambda b, s: (0, 0)),
        ]
        args += [fdp, W2a, W2b, b2p]
        HoutP = H3P
    else:
        HoutP = H2P

    def kern(*refs):
        _conv_kern(N, SB, r2, nsample, nW1, has_fd, refs)

    out = _call(
        kern,
        grid=grid,
        in_specs=in_specs,
        out_specs=pl.BlockSpec((1, SB, HoutP), lambda b, s: (b, s, 0)),
        out_shape=jax.ShapeDtypeStruct((B, S, HoutP), jnp.float32),
    )(*args)
    return out


def _down0_kern(x_ref, W_ref, b_ref, o_ref):
    o_ref[0] = jnp.maximum(
        jax.lax.dot_general(x_ref[0], W_ref[...], (((1,), (0,)), ((), ())),
                            preferred_element_type=jnp.float32) + b_ref[...],
        0.0)


def _down0(feat, W, b):
    B, N, C = feat.shape
    H = W.shape[1]
    HP = _rup(H, 128)
    xp = _pad_last(feat, 8)
    Wp = _pad_last(jnp.pad(W, ((0, 8 - C), (0, 0))), HP)
    bp = _pad_last(b[None, :], HP)
    out = _call(
        _down0_kern,
        grid=(B,),
        in_specs=[pl.BlockSpec((1, N, 8), lambda bb: (bb, 0, 0)),
                  pl.BlockSpec(Wp.shape, lambda bb: (0, 0)),
                  pl.BlockSpec(bp.shape, lambda bb: (0, 0))],
        out_specs=pl.BlockSpec((1, N, HP), lambda bb: (bb, 0, 0)),
        out_shape=jax.ShapeDtypeStruct((B, N, HP), jnp.float32),
    )(xp, Wp, bp)
    return out


def kernel(pc, feat, down0_W, down0_b, d1_W0, d1_b0, d1_W1, d1_b1,
           d2_W0, d2_b0, d2_W1, d2_b1, d3_W0, d3_b0, d3_W1, d3_b1,
           d4_W0, d4_b0, d4_W1, d4_b1, u4_W1, u4_b1, u4_W2, u4_b2,
           u3_W1, u3_b1, u3_W2, u3_b2, u2_W1, u2_b1, u2_W2, u2_b2,
           u1_W1, u1_b1, u1_W2, u1_b2):
    f0p = _down0(feat, down0_W, down0_b)
    f0 = f0p[:, :, :32]

    i1, x1 = _fps(pc, 2048)
    f1 = _conv_level(pc, x1, f0, 0.5, 16, [d1_W0, d1_W1], [d1_b0, d1_b1],
                     SB=128)[:, :, :64]
    i2, x2 = _fps(x1, 512)
    f2 = _conv_level(x1, x2, f1, 1.0, 16, [d2_W0, d2_W1], [d2_b0, d2_b1],
                     SB=512)[:, :, :128]
    i3, x3 = _fps(x2, 128)
    f3 = _conv_level(x2, x3, f2, 2.0, 16, [d3_W0, d3_W1], [d3_b0, d3_b1],
                     SB=128)[:, :, :192]
    i4, x4 = _fps(x3, 64)
    f4 = _conv_level(x3, x4, f3, 4.0, 16, [d4_W0, d4_W1], [d4_b0, d4_b1],
                     SB=64)[:, :, :192]

    u3 = _conv_level(x4, x3, f4, 6.0, 8, [u4_W1], [u4_b1], SB=128,
                     feat_d=f3, W2=u4_W2, b2=u4_b2)[:, :, :192]
    u2 = _conv_level(x3, x2, u3, 3.0, 8, [u3_W1], [u3_b1], SB=512,
                     feat_d=f2, W2=u3_W2, b2=u3_b2)[:, :, :128]
    u1 = _conv_level(x2, x1, u2, 1.5, 8, [u2_W1], [u2_b1], SB=512,
                     feat_d=f1, W2=u2_W2, b2=u2_b2)[:, :, :64]
    u0 = _conv_level(x1, pc, u1, 0.75, 8, [u1_W1], [u1_b1], SB=512,
                     feat_d=f0, W2=u1_W2, b2=u1_b2)[:, :, :32]

    return (x1, x2, x3, x4), (i1, i2, i3, i4), (u0, u1, u2, u3)


# speedup vs baseline: 14.2568x; 1.4546x over previous
import functools

import jax
import jax.numpy as jnp
from jax.experimental import pallas as pl
from jax.experimental.pallas import tpu as pltpu


def _call(*args, **kwargs):
    return pl.pallas_call(*args, **kwargs)


def _rup(x, m):
    return ((x + m - 1) // m) * m


def _pad_last(a, to):
    if a.shape[-1] == to:
        return a
    pad = [(0, 0)] * (a.ndim - 1) + [(0, to - a.shape[-1])]
    return jnp.pad(a, pad)


def _fps_kern(npoint, N, NL, x_ref, xr_ref, idx_ref, nx_ref, dists):
    X = x_ref[0]
    Y = x_ref[1]
    Z = x_ref[2]
    iota2 = (jnp.remainder(
        jax.lax.broadcasted_iota(jnp.int32, (16, NL), 0), 8) * NL
        + jax.lax.broadcasted_iota(jnp.int32, (16, NL), 1))
    dists[...] = jnp.full((16, NL), 1e10, jnp.float32)

    def step(t, fars):
        far0, far1 = fars
        crow0 = xr_ref[0, pl.ds(far0, 1), :]
        crow1 = xr_ref[1, pl.ds(far1, 1), :]
        idx_ref[0, pl.ds(t, 1), :] = jnp.full((1, 128), far0, jnp.int32)
        idx_ref[1, pl.ds(t, 1), :] = jnp.full((1, 128), far1, jnp.int32)
        nx_ref[0, pl.ds(t, 1), :] = crow0
        nx_ref[1, pl.ds(t, 1), :] = crow1
        cx = jnp.concatenate([jnp.broadcast_to(crow0[0:1, 0:1], (8, 1)),
                              jnp.broadcast_to(crow1[0:1, 0:1], (8, 1))], 0)
        cy = jnp.concatenate([jnp.broadcast_to(crow0[0:1, 1:2], (8, 1)),
                              jnp.broadcast_to(crow1[0:1, 1:2], (8, 1))], 0)
        cz = jnp.concatenate([jnp.broadcast_to(crow0[0:1, 2:3], (8, 1)),
                              jnp.broadcast_to(crow1[0:1, 2:3], (8, 1))], 0)
        dx = X - cx
        dy = Y - cy
        dz = Z - cz
        d = (dx * dx + dy * dy) + dz * dz
        nd = jnp.minimum(dists[...], d)
        dists[...] = nd
        nd0 = nd[0:8]
        nd1 = nd[8:16]
        m0 = jnp.max(nd0)
        m1 = jnp.max(nd1)
        io8 = iota2[0:8]
        nf0 = jnp.min(jnp.where(nd0 == m0, io8, N)).astype(jnp.int32)
        nf1 = jnp.min(jnp.where(nd1 == m1, io8, N)).astype(jnp.int32)
        return (nf0, nf1)

    jax.lax.fori_loop(0, npoint, step, (jnp.int32(0), jnp.int32(0)))


def _fps(xyz, npoint):
    B, N, _ = xyz.shape
    NL = N // 8
    xr = xyz.transpose(2, 0, 1).reshape(3, B * 8, NL)
    xrows = _pad_last(xyz, 128)
    idx, nx = _call(
        functools.partial(_fps_kern, npoint, N, NL),
        out_shape=[jax.ShapeDtypeStruct((B, npoint, 128), jnp.int32),
                   jax.ShapeDtypeStruct((B, npoint, 128), jnp.float32)],
        scratch_shapes=[pltpu.VMEM((16, NL), jnp.float32)],
    )(xr, xrows)
    return idx[:, :, 0], nx[:, :, :3]


def _conv_kern(N, SB, r2, ns, nW1, has_fd, refs):
    if has_fd:
        (xyzT_ref, q_ref, F_ref, W0_ref, b0_ref, W1_ref, b1_ref,
         fd_ref, W2a_ref, W2b_ref, b2_ref, o_ref) = refs
    else:
        (xyzT_ref, q_ref, F_ref, W0_ref, b0_ref, W1_ref, b1_ref,
         o_ref) = refs
    q = q_ref[0]
    xx = xyzT_ref[0, 0:1, :]
    xy = xyzT_ref[0, 1:2, :]
    xz = xyzT_ref[0, 2:3, :]
    dx = q[:, 0:1] - xx
    dy = q[:, 1:2] - xy
    dz = q[:, 2:3] - xz
    sq = (dx * dx + dy * dy) + dz * dz
    ioN = jax.lax.broadcasted_iota(jnp.int32, (SB, N), 1)
    mask = sq <= r2
    rank = mask.astype(jnp.int32)
    k = 1
    while k < N:
        rolled = pltpu.roll(rank, k, axis=1)
        rank = rank + jnp.where(ioN >= k, rolled, 0)
        k *= 2
    cnt = rank[:, N - 1:N]
    ohz = (cnt == 0) & (ioN == N - 1)
    Fb = F_ref[0]
    W0 = W0_ref[...]
    b0 = b0_ref[...]
    W1 = W1_ref[...]
    b1 = b1_ref[...]
    acc = None
    for j in range(ns):
        selj = jnp.where(cnt >= j + 1, j + 1, 1)
        ohj = ((mask & (rank == selj)) | ohz).astype(jnp.float32)
        g = jax.lax.dot_general(ohj, Fb, (((1,), (0,)), ((), ())),
                                preferred_element_type=jnp.float32)
        g = g - q
        h = jnp.maximum(
            jax.lax.dot_general(g, W0, (((1,), (0,)), ((), ())),
                                preferred_element_type=jnp.float32) + b0, 0.0)
        if nW1:
            h = jnp.maximum(
                jax.lax.dot_general(h, W1, (((1,), (0,)), ((), ())),
                                    preferred_element_type=jnp.float32) + b1,
                0.0)
        acc = h if acc is None else jnp.maximum(acc, h)
    if has_fd:
        fd = fd_ref[0]
        o = jnp.maximum(
            jax.lax.dot_general(acc, W2a_ref[...], (((1,), (0,)), ((), ())),
                                preferred_element_type=jnp.float32)
            + jax.lax.dot_general(fd, W2b_ref[...], (((1,), (0,)), ((), ())),
                                  preferred_element_type=jnp.float32)
            + b2_ref[...], 0.0)
    else:
        o = acc
    o_ref[0] = o


def _conv_level(xyz, new_xyz, feat, radius, nsample, Ws, bs, SB,
                feat_d=None, W2=None, b2=None):
    B, N, _ = xyz.shape
    S = new_xyz.shape[1]
    Cf = feat.shape[-1]
    Cin = 3 + Cf
    CP = _rup(Cin, 128)
    H1 = Ws[0].shape[1]
    H1P = _rup(H1, 128)
    nW1 = len(Ws) > 1
    if nW1:
        H2 = Ws[1].shape[1]
        H2P = _rup(H2, 128)
    else:
        H2, H2P = H1, H1P
    has_fd = feat_d is not None

    xyzT = _pad_last(xyz, 8).transpose(0, 2, 1)
    q_pad = _pad_last(new_xyz, CP)
    F_all = _pad_last(jnp.concatenate([xyz, feat], -1), CP)
    W0p = _pad_last(jnp.pad(Ws[0], ((0, CP - Cin), (0, 0))), H1P)
    b0p = _pad_last(bs[0][None, :], H1P)
    if nW1:
        W1p = _pad_last(jnp.pad(Ws[1], ((0, H1P - H1), (0, 0))), H2P)
        b1p = _pad_last(bs[1][None, :], H2P)
    else:
        W1p = jnp.zeros((8, 128), jnp.float32)
        b1p = jnp.zeros((1, 128), jnp.float32)

    r2 = float(radius) * float(radius)
    nsb = S // SB
    grid = (B, nsb)
    in_specs = [
        pl.BlockSpec((1, 8, N), lambda b, s: (b, 0, 0)),
        pl.BlockSpec((1, SB, CP), lambda b, s: (b, s, 0)),
        pl.BlockSpec((1, N, CP), lambda b, s: (b, 0, 0)),
        pl.BlockSpec(W0p.shape, lambda b, s: (0, 0)),
        pl.BlockSpec(b0p.shape, lambda b, s: (0, 0)),
        pl.BlockSpec(W1p.shape, lambda b, s: (0, 0)),
        pl.BlockSpec(b1p.shape, lambda b, s: (0, 0)),
    ]
    args = [xyzT, q_pad, F_all, W0p, b0p, W1p, b1p]
    if has_fd:
        Cd = feat_d.shape[-1]
        CdP = _rup(Cd, 128)
        H3 = W2.shape[1]
        H3P = _rup(H3, 128)
        fdp = _pad_last(feat_d, CdP)
        W2a = _pad_last(jnp.pad(W2[:H2], ((0, H2P - H2), (0, 0))), H3P)
        W2b = _pad_last(jnp.pad(W2[H2:], ((0, CdP - Cd), (0, 0))), H3P)
        b2p = _pad_last(b2[None, :], H3P)
        in_specs += [
            pl.BlockSpec((1, SB, CdP), lambda b, s: (b, s, 0)),
            pl.BlockSpec(W2a.shape, lambda b, s: (0, 0)),
            pl.BlockSpec(W2b.shape, lambda b, s: (0, 0)),
            pl.BlockSpec(b2p.shape, lambda b, s: (0, 0)),
        ]
        args += [fdp, W2a, W2b, b2p]
        HoutP = H3P
    else:
        HoutP = H2P

    def kern(*refs):
        _conv_kern(N, SB, r2, nsample, nW1, has_fd, refs)

    out = _call(
        kern,
        grid=grid,
        in_specs=in_specs,
        out_specs=pl.BlockSpec((1, SB, HoutP), lambda b, s: (b, s, 0)),
        out_shape=jax.ShapeDtypeStruct((B, S, HoutP), jnp.float32),
    )(*args)
    return out


def _down0_kern(x_ref, W_ref, b_ref, o_ref):
    o_ref[0] = jnp.maximum(
        jax.lax.dot_general(x_ref[0], W_ref[...], (((1,), (0,)), ((), ())),
                            preferred_element_type=jnp.float32) + b_ref[...],
        0.0)


def _down0(feat, W, b):
    B, N, C = feat.shape
    H = W.shape[1]
    HP = _rup(H, 128)
    xp = _pad_last(feat, 8)
    Wp = _pad_last(jnp.pad(W, ((0, 8 - C), (0, 0))), HP)
    bp = _pad_last(b[None, :], HP)
    out = _call(
        _down0_kern,
        grid=(B,),
        in_specs=[pl.BlockSpec((1, N, 8), lambda bb: (bb, 0, 0)),
                  pl.BlockSpec(Wp.shape, lambda bb: (0, 0)),
                  pl.BlockSpec(bp.shape, lambda bb: (0, 0))],
        out_specs=pl.BlockSpec((1, N, HP), lambda bb: (bb, 0, 0)),
        out_shape=jax.ShapeDtypeStruct((B, N, HP), jnp.float32),
    )(xp, Wp, bp)
    return out


def kernel(pc, feat, down0_W, down0_b, d1_W0, d1_b0, d1_W1, d1_b1,
           d2_W0, d2_b0, d2_W1, d2_b1, d3_W0, d3_b0, d3_W1, d3_b1,
           d4_W0, d4_b0, d4_W1, d4_b1, u4_W1, u4_b1, u4_W2, u4_b2,
           u3_W1, u3_b1, u3_W2, u3_b2, u2_W1, u2_b1, u2_W2, u2_b2,
           u1_W1, u1_b1, u1_W2, u1_b2):
    f0p = _down0(feat, down0_W, down0_b)
    f0 = f0p[:, :, :32]

    i1, x1 = _fps(pc, 2048)
    f1 = _conv_level(pc, x1, f0, 0.5, 16, [d1_W0, d1_W1], [d1_b0, d1_b1],
                     SB=128)[:, :, :64]
    i2, x2 = _fps(x1, 512)
    f2 = _conv_level(x1, x2, f1, 1.0, 16, [d2_W0, d2_W1], [d2_b0, d2_b1],
                     SB=512)[:, :, :128]
    i3, x3 = _fps(x2, 128)
    f3 = _conv_level(x2, x3, f2, 2.0, 16, [d3_W0, d3_W1], [d3_b0, d3_b1],
                     SB=128)[:, :, :192]
    i4, x4 = _fps(x3, 64)
    f4 = _conv_level(x3, x4, f3, 4.0, 16, [d4_W0, d4_W1], [d4_b0, d4_b1],
                     SB=64)[:, :, :192]

    u3 = _conv_level(x4, x3, f4, 6.0, 8, [u4_W1], [u4_b1], SB=128,
                     feat_d=f3, W2=u4_W2, b2=u4_b2)[:, :, :192]
    u2 = _conv_level(x3, x2, u3, 3.0, 8, [u3_W1], [u3_b1], SB=512,
                     feat_d=f2, W2=u3_W2, b2=u3_b2)[:, :, :128]
    u1 = _conv_level(x2, x1, u2, 1.5, 8, [u2_W1], [u2_b1], SB=512,
                     feat_d=f1, W2=u2_W2, b2=u2_b2)[:, :, :64]
    u0 = _conv_level(x1, pc, u1, 0.75, 8, [u1_W1], [u1_b1], SB=512,
                     feat_d=f0, W2=u1_W2, b2=u1_b2)[:, :, :32]

    return (x1, x2, x3, x4), (i1, i2, i3, i4), (u0, u1, u2, u3)
